# baseline (speedup 1.0000x reference)
import functools

import jax
import jax.numpy as jnp
from jax import lax
from jax.experimental import pallas as pl
from jax.experimental.pallas import tpu as pltpu
from jax.experimental.pallas import tpu_sc as plsc

D = 128
LANG_PAD = 104


def _project_body(char_ref, lang_ref, w_ref, b_ref, cout_ref, lout_ref):
    w = w_ref[...]
    w1 = w[:, :D]
    w2 = w[:, D:]
    cn = (((1,), (1,)), ((), ()))
    cout_ref[...] = (
        lax.dot_general(char_ref[...], w1, cn, preferred_element_type=jnp.float32)
        + b_ref[...]
    )
    lout_ref[...] = lax.dot_general(
        lang_ref[...], w2, cn, preferred_element_type=jnp.float32
    )


def _project(char_table, lang_table_padded, W, b2d):
    n_chars = char_table.shape[0]
    return pl.pallas_call(
        _project_body,
        out_shape=[
            jax.ShapeDtypeStruct((n_chars, D), jnp.float32),
            jax.ShapeDtypeStruct((LANG_PAD, D), jnp.float32),
        ],
    )(char_table, lang_table_padded, W, b2d)


def _make_sc_gather(n_total):
    info = plsc.get_sparse_core_info()
    nw = info.num_cores * info.num_subcores
    per_w = n_total // nw
    ch = 128
    n_ch = per_w // ch
    mesh = plsc.VectorSubcoreMesh(core_axis_name="c", subcore_axis_name="s")

    @functools.partial(
        pl.kernel,
        mesh=mesh,
        out_type=jax.ShapeDtypeStruct((n_total, D), jnp.float32),
        scratch_types=[
            pltpu.VMEM((per_w,), jnp.int32),
            pltpu.VMEM((per_w,), jnp.int32),
            pltpu.VMEM((ch, D), jnp.float32),
            pltpu.VMEM((ch, D), jnp.float32),
            pltpu.VMEM((ch, D), jnp.float32),
            pltpu.VMEM((ch, D), jnp.float32),
            pltpu.VMEM((ch, D), jnp.float32),
            pltpu.VMEM((ch, D), jnp.float32),
            pltpu.SemaphoreType.DMA,
            pltpu.SemaphoreType.DMA,
            pltpu.SemaphoreType.DMA,
            pltpu.SemaphoreType.DMA,
            pltpu.SemaphoreType.DMA,
            pltpu.SemaphoreType.DMA,
        ],
    )
    def sc_gather(cproj_hbm, lproj_hbm, ci_hbm, li_hbm, out_hbm,
                  ci_v, li_v, ga0, ga1, la0, la1, ob0, ob1,
                  gs0, gs1, ls0, ls1, ss0, ss1):
        wid = lax.axis_index("s") * info.num_cores + lax.axis_index("c")
        base = wid * per_w
        pltpu.sync_copy(ci_hbm.at[pl.ds(base, per_w)], ci_v)
        pltpu.sync_copy(li_hbm.at[pl.ds(base, per_w)], li_v)
        gbufs = (ga0, ga1)
        lbufs = (la0, la1)
        obufs = (ob0, ob1)
        gsems = (gs0, gs1)
        lsems = (ls0, ls1)
        ssems = (ss0, ss1)

        def start_gather(g, b):
            pltpu.async_copy(
                cproj_hbm.at[ci_v.at[pl.ds(g * ch, ch)]], gbufs[b], gsems[b]
            )
            pltpu.async_copy(
                lproj_hbm.at[li_v.at[pl.ds(g * ch, ch)]], lbufs[b], lsems[b]
            )

        def wait_gather(b):
            pltpu.make_async_copy(
                cproj_hbm.at[ci_v.at[pl.ds(0, ch)]], gbufs[b], gsems[b]
            ).wait()
            pltpu.make_async_copy(
                lproj_hbm.at[li_v.at[pl.ds(0, ch)]], lbufs[b], lsems[b]
            ).wait()

        def start_scatter(g, b):
            return pltpu.async_copy(
                obufs[b], out_hbm.at[pl.ds(base + g * ch, ch)], ssems[b]
            )

        def wait_scatter(b):
            pltpu.make_async_copy(
                obufs[b], out_hbm.at[pl.ds(base, ch)], ssems[b]
            ).wait()

        def add_chunk(g, b):
            gb = gbufs[b]
            lb = lbufs[b]
            ob = obufs[b]

            def grp(t, carry):
                r0 = t * 16
                for j in range(16):
                    for c in range(0, D, 16):
                        ob[r0 + j, pl.ds(c, 16)] = (
                            gb[r0 + j, pl.ds(c, 16)] + lb[r0 + j, pl.ds(c, 16)]
                        )
                return carry

            lax.fori_loop(0, ch // 16, grp, 0)

        start_gather(0, 0)
        start_gather(1, 1)
        for g in (0, 1):
            b = g % 2
            wait_gather(b)
            add_chunk(g, b)
            start_scatter(g, b)
            start_gather(g + 2, b)

        def pair(i, carry):
            g = 2 + 2 * i
            for b in (0, 1):
                wait_gather(b)
                wait_scatter(b)
                add_chunk(g + b, b)
                start_scatter(g + b, b)
                start_gather(g + b + 2, b)
            return carry

        n_pairs = (n_ch - 4) // 2
        lax.fori_loop(0, n_pairs, pair, 0)

        for g in (n_ch - 2, n_ch - 1):
            b = g % 2
            wait_gather(b)
            wait_scatter(b)
            add_chunk(g, b)
            start_scatter(g, b)
        for b in (0, 1):
            wait_scatter(b)

    return sc_gather


def kernel(char_indices, lang_indices, char_table, lang_table, W, b):
    B, L = char_indices.shape
    n_total = B * L
    lang_padded = jnp.pad(lang_table, ((0, LANG_PAD - lang_table.shape[0]), (0, 0)))
    cproj, lproj = _project(char_table, lang_padded, W, b.reshape(1, D))
    ci = char_indices.reshape(-1).astype(jnp.int32)
    li = lang_indices.reshape(-1).astype(jnp.int32)
    out = _make_sc_gather(n_total)(cproj, lproj, ci, li)
    return out.reshape(B, L, D)


# speedup vs baseline: 1.0025x; 1.0025x over previous
import functools

import jax
import jax.numpy as jnp
from jax import lax
from jax.experimental import pallas as pl
from jax.experimental.pallas import tpu as pltpu
from jax.experimental.pallas import tpu_sc as plsc

D = 128
LANG_PAD = 104


def _project_body(char_ref, lang_ref, w_ref, b_ref, cout_ref, lout_ref):
    w = w_ref[...]
    w1 = w[:, :D]
    w2 = w[:, D:]
    cn = (((1,), (1,)), ((), ()))
    cout_ref[...] = (
        lax.dot_general(char_ref[...], w1, cn, preferred_element_type=jnp.float32)
        + b_ref[...]
    )
    lout_ref[...] = lax.dot_general(
        lang_ref[...], w2, cn, preferred_element_type=jnp.float32
    )


def _project(char_table, lang_table_padded, W, b2d):
    n_chars = char_table.shape[0]
    return pl.pallas_call(
        _project_body,
        out_shape=[
            jax.ShapeDtypeStruct((n_chars, D), jnp.float32),
            jax.ShapeDtypeStruct((LANG_PAD, D), jnp.float32),
        ],
    )(char_table, lang_table_padded, W, b2d)


def _make_sc_gather(n_total):
    info = plsc.get_sparse_core_info()
    nw = info.num_cores * info.num_subcores
    per_w = n_total // nw
    ch = 128
    n_ch = per_w // ch
    mesh = plsc.VectorSubcoreMesh(core_axis_name="c", subcore_axis_name="s")

    @functools.partial(
        pl.kernel,
        mesh=mesh,
        out_type=jax.ShapeDtypeStruct((n_total, D), jnp.float32),
        scratch_types=[
            pltpu.VMEM((per_w,), jnp.int32),
            pltpu.VMEM((per_w,), jnp.int32),
            pltpu.VMEM((ch, D), jnp.float32),
            pltpu.VMEM((ch, D), jnp.float32),
            pltpu.VMEM((ch, D), jnp.float32),
            pltpu.SemaphoreType.DMA,
            pltpu.SemaphoreType.DMA,
            pltpu.SemaphoreType.DMA,
            pltpu.SemaphoreType.DMA,
            pltpu.SemaphoreType.DMA,
            pltpu.SemaphoreType.DMA,
            pltpu.SemaphoreType.DMA,
            pltpu.SemaphoreType.DMA,
            pltpu.SemaphoreType.DMA,
        ],
    )
    def sc_gather(cproj_hbm, lproj_hbm, ci_hbm, li_hbm, out_hbm,
                  ci_v, li_v, ga0, ga1, ga2,
                  gs0, gs1, gs2, ls0, ls1, ls2, ss0, ss1, ss2):
        wid = lax.axis_index("s") * info.num_cores + lax.axis_index("c")
        base = wid * per_w
        pltpu.sync_copy(ci_hbm.at[pl.ds(base, per_w)], ci_v)
        pltpu.sync_copy(li_hbm.at[pl.ds(base, per_w)], li_v)
        gbufs = (ga0, ga1, ga2)
        gsems = (gs0, gs1, gs2)
        lsems = (ls0, ls1, ls2)
        ssems = (ss0, ss1, ss2)

        def start_gather(g, b):
            pltpu.async_copy(
                cproj_hbm.at[ci_v.at[pl.ds(g * ch, ch)]], gbufs[b], gsems[b]
            )

        def start_lang_add(g, b):
            pltpu.async_copy(
                lproj_hbm.at[li_v.at[pl.ds(g * ch, ch)]], gbufs[b], lsems[b],
                add=True,
            )

        def wait_gather(b):
            pltpu.make_async_copy(
                cproj_hbm.at[ci_v.at[pl.ds(0, ch)]], gbufs[b], gsems[b]
            ).wait()

        def wait_lang_add(b):
            pltpu.make_async_copy(
                lproj_hbm.at[li_v.at[pl.ds(0, ch)]], gbufs[b], lsems[b]
            ).wait()

        def start_scatter(g, b):
            return pltpu.async_copy(
                gbufs[b], out_hbm.at[pl.ds(base + g * ch, ch)], ssems[b]
            )

        def wait_scatter(b):
            pltpu.make_async_copy(
                gbufs[b], out_hbm.at[pl.ds(base, ch)], ssems[b]
            ).wait()

        start_gather(0, 0)
        start_gather(1, 1)
        for g in range(n_ch):
            b = g % 3
            bn = (g + 2) % 3
            if g + 2 < n_ch:
                if g >= 1:
                    wait_scatter(bn)
                start_gather(g + 2, bn)
            wait_gather(b)
            start_lang_add(g, b)
            wait_lang_add(b)
            start_scatter(g, b)
        for b in range(3):
            wait_scatter(b)

    return sc_gather


def kernel(char_indices, lang_indices, char_table, lang_table, W, b):
    B, L = char_indices.shape
    n_total = B * L
    lang_padded = jnp.pad(lang_table, ((0, LANG_PAD - lang_table.shape[0]), (0, 0)))
    cproj, lproj = _project(char_table, lang_padded, W, b.reshape(1, D))
    ci = char_indices.reshape(-1).astype(jnp.int32)
    li = lang_indices.reshape(-1).astype(jnp.int32)
    out = _make_sc_gather(n_total)(cproj, lproj, ci, li)
    return out.reshape(B, L, D)


# speedup vs baseline: 1.0026x; 1.0001x over previous
import functools

import jax
import jax.numpy as jnp
from jax import lax
from jax.experimental import pallas as pl
from jax.experimental.pallas import tpu as pltpu
from jax.experimental.pallas import tpu_sc as plsc

D = 128
LANG_PAD = 104


def _project_body(char_ref, lang_ref, w_ref, b_ref, cout_ref, lout_ref):
    w = w_ref[...]
    w1 = w[:, :D]
    w2 = w[:, D:]
    cn = (((1,), (1,)), ((), ()))
    cout_ref[...] = (
        lax.dot_general(char_ref[...], w1, cn, preferred_element_type=jnp.float32)
        + b_ref[...]
    )
    lout_ref[...] = lax.dot_general(
        lang_ref[...], w2, cn, preferred_element_type=jnp.float32
    )


def _project(char_table, lang_table_padded, W, b2d):
    n_chars = char_table.shape[0]
    return pl.pallas_call(
        _project_body,
        out_shape=[
            jax.ShapeDtypeStruct((n_chars, D), jnp.float32),
            jax.ShapeDtypeStruct((LANG_PAD, D), jnp.float32),
        ],
    )(char_table, lang_table_padded, W, b2d)


def _make_sc_gather(n_total):
    info = plsc.get_sparse_core_info()
    nw = info.num_cores * info.num_subcores
    per_w = n_total // nw
    ch = 128
    n_ch = per_w // ch
    mesh = plsc.VectorSubcoreMesh(core_axis_name="c", subcore_axis_name="s")

    @functools.partial(
        pl.kernel,
        mesh=mesh,
        out_type=jax.ShapeDtypeStruct((n_total, D), jnp.float32),
        scratch_types=[
            pltpu.VMEM((per_w,), jnp.int32),
            pltpu.VMEM((per_w,), jnp.int32),
            pltpu.VMEM((ch, D), jnp.float32),
            pltpu.VMEM((ch, D), jnp.float32),
            pltpu.VMEM((ch, D), jnp.float32),
            pltpu.VMEM((ch, D), jnp.float32),
            pltpu.VMEM((ch, D), jnp.float32),
            pltpu.VMEM((ch, D), jnp.float32),
            pltpu.SemaphoreType.DMA,
            pltpu.SemaphoreType.DMA,
            pltpu.SemaphoreType.DMA,
            pltpu.SemaphoreType.DMA,
            pltpu.SemaphoreType.DMA,
            pltpu.SemaphoreType.DMA,
            pltpu.SemaphoreType.DMA,
            pltpu.SemaphoreType.DMA,
            pltpu.SemaphoreType.DMA,
            pltpu.SemaphoreType.DMA,
            pltpu.SemaphoreType.DMA,
            pltpu.SemaphoreType.DMA,
            pltpu.SemaphoreType.DMA,
            pltpu.SemaphoreType.DMA,
            pltpu.SemaphoreType.DMA,
            pltpu.SemaphoreType.DMA,
            pltpu.SemaphoreType.DMA,
            pltpu.SemaphoreType.DMA,
        ],
    )
    def sc_gather(cproj_hbm, lproj_hbm, ci_hbm, li_hbm, out_hbm,
                  ci_v, li_v, ga0, ga1, ga2, ga3, ga4, ga5,
                  gs0, gs1, gs2, gs3, gs4, gs5,
                  ls0, ls1, ls2, ls3, ls4, ls5,
                  ss0, ss1, ss2, ss3, ss4, ss5):
        wid = lax.axis_index("s") * info.num_cores + lax.axis_index("c")
        base = wid * per_w
        pltpu.sync_copy(ci_hbm.at[pl.ds(base, per_w)], ci_v)
        pltpu.sync_copy(li_hbm.at[pl.ds(base, per_w)], li_v)
        gbufs = (ga0, ga1, ga2, ga3, ga4, ga5)
        gsems = (gs0, gs1, gs2, gs3, gs4, gs5)
        lsems = (ls0, ls1, ls2, ls3, ls4, ls5)
        ssems = (ss0, ss1, ss2, ss3, ss4, ss5)
        nbuf = 6

        def start_gather(g, b):
            pltpu.async_copy(
                cproj_hbm.at[ci_v.at[pl.ds(g * ch, ch)]], gbufs[b], gsems[b]
            )

        def start_lang_add(g, b):
            pltpu.async_copy(
                lproj_hbm.at[li_v.at[pl.ds(g * ch, ch)]], gbufs[b], lsems[b],
                add=True,
            )

        def wait_gather(b):
            pltpu.make_async_copy(
                cproj_hbm.at[ci_v.at[pl.ds(0, ch)]], gbufs[b], gsems[b]
            ).wait()

        def wait_lang_add(b):
            pltpu.make_async_copy(
                lproj_hbm.at[li_v.at[pl.ds(0, ch)]], gbufs[b], lsems[b]
            ).wait()

        def start_scatter(g, b):
            return pltpu.async_copy(
                gbufs[b], out_hbm.at[pl.ds(base + g * ch, ch)], ssems[b]
            )

        def wait_scatter(b):
            pltpu.make_async_copy(
                gbufs[b], out_hbm.at[pl.ds(base, ch)], ssems[b]
            ).wait()

        for g in range(4):
            start_gather(g, g % nbuf)
        wait_gather(0)
        start_lang_add(0, 0)
        for t in range(n_ch):
            if t + 4 < n_ch:
                if t - 2 >= 0:
                    wait_scatter((t + 4) % nbuf)
                start_gather(t + 4, (t + 4) % nbuf)
            if t + 1 < n_ch:
                wait_gather((t + 1) % nbuf)
                start_lang_add(t + 1, (t + 1) % nbuf)
            wait_lang_add(t % nbuf)
            start_scatter(t, t % nbuf)
        for b in range(nbuf):
            wait_scatter(b)

    return sc_gather


def kernel(char_indices, lang_indices, char_table, lang_table, W, b):
    B, L = char_indices.shape
    n_total = B * L
    lang_padded = jnp.pad(lang_table, ((0, LANG_PAD - lang_table.shape[0]), (0, 0)))
    cproj, lproj = _project(char_table, lang_padded, W, b.reshape(1, D))
    ci = char_indices.reshape(-1).astype(jnp.int32)
    li = lang_indices.reshape(-1).astype(jnp.int32)
    out = _make_sc_gather(n_total)(cproj, lproj, ci, li)
    return out.reshape(B, L, D)


# speedup vs baseline: 1.1656x; 1.1626x over previous
import functools

import jax
import jax.numpy as jnp
from jax import lax
from jax.experimental import pallas as pl
from jax.experimental.pallas import tpu as pltpu
from jax.experimental.pallas import tpu_sc as plsc

D = 128
LANG_PAD = 104


def _project_body(char_ref, lang_ref, w_ref, b_ref, cout_ref, lout_ref):
    w = w_ref[...]
    w1 = w[:, :D]
    w2 = w[:, D:]
    cn = (((1,), (1,)), ((), ()))
    cout_ref[...] = (
        lax.dot_general(char_ref[...], w1, cn, preferred_element_type=jnp.float32)
        + b_ref[...]
    )
    lout_ref[...] = lax.dot_general(
        lang_ref[...], w2, cn, preferred_element_type=jnp.float32
    )


def _project(char_table, lang_table_padded, W, b2d):
    n_chars = char_table.shape[0]
    return pl.pallas_call(
        _project_body,
        out_shape=[
            jax.ShapeDtypeStruct((n_chars, D), jnp.float32),
            jax.ShapeDtypeStruct((LANG_PAD, D), jnp.float32),
        ],
    )(char_table, lang_table_padded, W, b2d)


def _make_sc_gather(n_total):
    info = plsc.get_sparse_core_info()
    nw = info.num_cores * info.num_subcores
    per_w = n_total // nw
    ch = 128
    n_ch = per_w // ch
    nbuf = 4
    mesh = plsc.VectorSubcoreMesh(core_axis_name="c", subcore_axis_name="s")

    @functools.partial(
        pl.kernel,
        mesh=mesh,
        out_type=jax.ShapeDtypeStruct((n_total, D), jnp.float32),
        scratch_types=[
            pltpu.VMEM((LANG_PAD, D), jnp.float32),
            pltpu.VMEM((per_w,), jnp.int32),
            pltpu.VMEM((per_w + 16,), jnp.int32),
            pltpu.VMEM((ch, D), jnp.float32),
            pltpu.VMEM((ch, D), jnp.float32),
            pltpu.VMEM((ch, D), jnp.float32),
            pltpu.VMEM((ch, D), jnp.float32),
            pltpu.SemaphoreType.DMA,
            pltpu.SemaphoreType.DMA,
            pltpu.SemaphoreType.DMA,
            pltpu.SemaphoreType.DMA,
            pltpu.SemaphoreType.DMA,
            pltpu.SemaphoreType.DMA,
            pltpu.SemaphoreType.DMA,
            pltpu.SemaphoreType.DMA,
        ],
    )
    def sc_gather(cproj_hbm, lproj_hbm, ci_hbm, li_hbm, out_hbm,
                  lang_v, ci_v, li_v, ga0, ga1, ga2, ga3,
                  gs0, gs1, gs2, gs3, ss0, ss1, ss2, ss3):
        wid = lax.axis_index("s") * info.num_cores + lax.axis_index("c")
        base = wid * per_w
        pltpu.sync_copy(lproj_hbm, lang_v)
        pltpu.sync_copy(ci_hbm.at[pl.ds(base, per_w)], ci_v)
        pltpu.sync_copy(li_hbm.at[pl.ds(base, per_w)], li_v.at[pl.ds(0, per_w)])
        gbufs = (ga0, ga1, ga2, ga3)
        gsems = (gs0, gs1, gs2, gs3)
        ssems = (ss0, ss1, ss2, ss3)

        def start_gather(g, b):
            pltpu.async_copy(
                cproj_hbm.at[ci_v.at[pl.ds(g * ch, ch)]], gbufs[b], gsems[b]
            )

        def wait_gather(b):
            pltpu.make_async_copy(
                cproj_hbm.at[ci_v.at[pl.ds(0, ch)]], gbufs[b], gsems[b]
            ).wait()

        def start_scatter(g, b):
            pltpu.async_copy(
                gbufs[b], out_hbm.at[pl.ds(base + g * ch, ch)], ssems[b]
            )

        def wait_scatter(b):
            pltpu.make_async_copy(
                gbufs[b], out_hbm.at[pl.ds(base, ch)], ssems[b]
            ).wait()

        def add_lang(g, b):
            buf = gbufs[b]
            lbase = g * ch

            def grp(t, carry):
                r0 = t * 4
                lvec = li_v[pl.ds(lbase + r0, 16)]
                for j in range(4):
                    lr = lvec[j]
                    for c in range(0, D, 16):
                        plsc.addupdate(
                            buf.at[r0 + j, pl.ds(c, 16)],
                            lang_v[lr, pl.ds(c, 16)],
                        )
                return carry

            lax.fori_loop(0, ch // 4, grp, 0)

        for g in range(3):
            start_gather(g, g % nbuf)
        for t in range(n_ch):
            b = t % nbuf
            if t + 3 < n_ch:
                bn = (t + 3) % nbuf
                if t - 1 >= 0:
                    wait_scatter(bn)
                start_gather(t + 3, bn)
            wait_gather(b)
            add_lang(t, b)
            start_scatter(t, b)
        for b in range(nbuf):
            wait_scatter(b)

    return sc_gather


def kernel(char_indices, lang_indices, char_table, lang_table, W, b):
    B, L = char_indices.shape
    n_total = B * L
    lang_padded = jnp.pad(lang_table, ((0, LANG_PAD - lang_table.shape[0]), (0, 0)))
    cproj, lproj = _project(char_table, lang_padded, W, b.reshape(1, D))
    ci = char_indices.reshape(-1).astype(jnp.int32)
    li = lang_indices.reshape(-1).astype(jnp.int32)
    out = _make_sc_gather(n_total)(cproj, lproj, ci, li)
    return out.reshape(B, L, D)


# speedup vs baseline: 1.4519x; 1.2457x over previous
import functools

import jax
import jax.numpy as jnp
from jax import lax
from jax.experimental import pallas as pl
from jax.experimental.pallas import tpu as pltpu
from jax.experimental.pallas import tpu_sc as plsc

D = 128
LANG_PAD = 104


def _project_body(char_ref, lang_ref, w_ref, b_ref, cout_ref, lout_ref):
    w = w_ref[...]
    w1 = w[:, :D]
    w2 = w[:, D:]
    cn = (((1,), (1,)), ((), ()))
    cout_ref[...] = (
        lax.dot_general(char_ref[...], w1, cn, preferred_element_type=jnp.float32)
        + b_ref[...]
    )
    lout_ref[...] = lax.dot_general(
        lang_ref[...], w2, cn, preferred_element_type=jnp.float32
    )


def _project(char_table, lang_table_padded, W, b2d):
    n_chars = char_table.shape[0]
    return pl.pallas_call(
        _project_body,
        out_shape=[
            jax.ShapeDtypeStruct((n_chars, D), jnp.float32),
            jax.ShapeDtypeStruct((LANG_PAD, D), jnp.float32),
        ],
    )(char_table, lang_table_padded, W, b2d)


def _make_sc_gather(n_total):
    info = plsc.get_sparse_core_info()
    nw = info.num_cores * info.num_subcores
    per_w = n_total // nw
    ch = 128
    n_ch = per_w // ch
    nbuf = 4
    mesh = plsc.VectorSubcoreMesh(core_axis_name="c", subcore_axis_name="s")

    @functools.partial(
        pl.kernel,
        mesh=mesh,
        out_type=jax.ShapeDtypeStruct((n_total, D), jnp.float32),
        scratch_types=[
            pltpu.VMEM((per_w,), jnp.int32),
            pltpu.VMEM((ch, D), jnp.float32),
            pltpu.VMEM((ch, D), jnp.float32),
            pltpu.VMEM((ch, D), jnp.float32),
            pltpu.VMEM((ch, D), jnp.float32),
            pltpu.SemaphoreType.DMA,
            pltpu.SemaphoreType.DMA,
            pltpu.SemaphoreType.DMA,
            pltpu.SemaphoreType.DMA,
            pltpu.SemaphoreType.DMA,
            pltpu.SemaphoreType.DMA,
            pltpu.SemaphoreType.DMA,
            pltpu.SemaphoreType.DMA,
        ],
    )
    def sc_gather(cproj_hbm, ci_hbm, out_hbm,
                  ci_v, ga0, ga1, ga2, ga3,
                  gs0, gs1, gs2, gs3, ss0, ss1, ss2, ss3):
        wid = lax.axis_index("s") * info.num_cores + lax.axis_index("c")
        base = wid * per_w
        pltpu.sync_copy(ci_hbm.at[pl.ds(base, per_w)], ci_v)
        gbufs = (ga0, ga1, ga2, ga3)
        gsems = (gs0, gs1, gs2, gs3)
        ssems = (ss0, ss1, ss2, ss3)

        def start_gather(g, b):
            pltpu.async_copy(
                cproj_hbm.at[ci_v.at[pl.ds(g * ch, ch)]], gbufs[b], gsems[b]
            )

        def wait_gather(b):
            pltpu.make_async_copy(
                cproj_hbm.at[ci_v.at[pl.ds(0, ch)]], gbufs[b], gsems[b]
            ).wait()

        def start_scatter(g, b):
            pltpu.async_copy(
                gbufs[b], out_hbm.at[pl.ds(base + g * ch, ch)], ssems[b]
            )

        def wait_scatter(b):
            pltpu.make_async_copy(
                gbufs[b], out_hbm.at[pl.ds(base, ch)], ssems[b]
            ).wait()

        for g in range(3):
            start_gather(g, g % nbuf)
        for t in range(n_ch):
            b = t % nbuf
            if t + 3 < n_ch:
                bn = (t + 3) % nbuf
                if t - 1 >= 0:
                    wait_scatter(bn)
                start_gather(t + 3, bn)
            wait_gather(b)
            start_scatter(t, b)
        for b in range(nbuf):
            wait_scatter(b)

    return sc_gather


_BB = 32


def _finish_body(part_ref, li_ref, lproj_ref, out_ref):
    L = li_ref.shape[1]
    li = li_ref[...]
    oh = (
        li[..., None] == lax.broadcasted_iota(jnp.int32, (_BB, L, LANG_PAD), 2)
    ).astype(jnp.float32)
    lang_rows = lax.dot_general(
        oh, lproj_ref[...], (((2,), (0,)), ((), ())),
        preferred_element_type=jnp.float32,
    )
    out_ref[...] = part_ref[...].reshape(_BB, L, D) + lang_rows


def _finish(flat, lang_indices, lproj):
    B, L = lang_indices.shape
    return pl.pallas_call(
        _finish_body,
        grid=(B // _BB,),
        in_specs=[
            pl.BlockSpec((_BB * L, D), lambda i: (i, 0)),
            pl.BlockSpec((_BB, L), lambda i: (i, 0)),
            pl.BlockSpec((LANG_PAD, D), lambda i: (0, 0)),
        ],
        out_specs=pl.BlockSpec((_BB, L, D), lambda i: (i, 0, 0)),
        out_shape=jax.ShapeDtypeStruct((B, L, D), jnp.float32),
    )(flat, lang_indices, lproj)


def kernel(char_indices, lang_indices, char_table, lang_table, W, b):
    B, L = char_indices.shape
    n_total = B * L
    lang_padded = jnp.pad(lang_table, ((0, LANG_PAD - lang_table.shape[0]), (0, 0)))
    cproj, lproj = _project(char_table, lang_padded, W, b.reshape(1, D))
    ci = char_indices.reshape(-1).astype(jnp.int32)
    out = _make_sc_gather(n_total)(cproj, ci)
    return _finish(out, lang_indices.astype(jnp.int32), lproj)
